# baseline (speedup 1.0000x reference)
import functools

import jax
import jax.numpy as jnp
import numpy as np
from jax import lax
from jax.experimental import pallas as pl
from jax.experimental.pallas import tpu as pltpu
from jax.experimental.pallas import tpu_sc as plsc

N = 10000
E = 320000
D = 128
DE = 16
H = 128
G = 64

NB = 10
BN = N // NB
EB = 40
BE = E // EB

NTILES = 32
EPT = E // NTILES
CH = 40
NCH = EPT // CH
SB = 624
TAIL = N - 16 * SB
ZR = 104


def _dot(a, b):
    return jnp.dot(a, b, preferred_element_type=jnp.float32)


def _edge_b1_body(ea_ref, c_ref, d_ref, b1_ref):
    b1_ref[...] = _dot(ea_ref[...], c_ref[...]) + d_ref[...]


_edge_b1 = pl.pallas_call(
    _edge_b1_body,
    grid=(EB,),
    in_specs=[
        pl.BlockSpec((BE, DE), lambda i: (i, 0)),
        pl.BlockSpec((DE, H), lambda i: (0, 0)),
        pl.BlockSpec((1, H), lambda i: (0, 0)),
    ],
    out_specs=pl.BlockSpec((BE, H), lambda i: (i, 0)),
    out_shape=jax.ShapeDtypeStruct((E, H), jnp.float32),
)


def _edge_b23_body(ea_ref, c_ref, d_ref, b2_ref, b3_ref):
    ea = ea_ref[...]
    b2_ref[...] = _dot(ea, c_ref[0]) + d_ref[0]
    b3_ref[...] = _dot(ea, c_ref[1]) + d_ref[1]


_edge_b23 = pl.pallas_call(
    _edge_b23_body,
    grid=(EB,),
    in_specs=[
        pl.BlockSpec((BE, DE), lambda i: (i, 0)),
        pl.BlockSpec((2, DE, H), lambda i: (0, 0, 0)),
        pl.BlockSpec((2, 1, H), lambda i: (0, 0, 0)),
    ],
    out_specs=[pl.BlockSpec((BE, H), lambda i: (i, 0))] * 2,
    out_shape=[jax.ShapeDtypeStruct((E, H), jnp.float32)] * 2,
)


def _pre_body(x_ref, q_ref, qb_ref, wu_ref, bu_ref, a_ref, px_ref):
    x = x_ref[...]
    a_ref[...] = _dot(x, q_ref[...]) + qb_ref[...]
    px_ref[...] = jnp.maximum(_dot(x, wu_ref[...]) + bu_ref[...], 0.0)


_pre = pl.pallas_call(
    _pre_body,
    grid=(NB,),
    in_specs=[
        pl.BlockSpec((BN, D), lambda i: (i, 0)),
        pl.BlockSpec((D, H), lambda i: (0, 0)),
        pl.BlockSpec((1, H), lambda i: (0, 0)),
        pl.BlockSpec((D, H), lambda i: (0, 0)),
        pl.BlockSpec((1, H), lambda i: (0, 0)),
    ],
    out_specs=[pl.BlockSpec((BN, H), lambda i: (i, 0))] * 2,
    out_shape=[jax.ShapeDtypeStruct((N, H), jnp.float32)] * 2,
)


def _mid_body(px_ref, agg_ref, wpt_ref, wpb_ref, bp_ref,
              q_ref, qb_ref, wu_ref, bu_ref, a_ref, pxn_ref):
    agg = agg_ref[0] + agg_ref[1]
    xn = jnp.maximum(
        _dot(px_ref[...], wpt_ref[...]) + _dot(agg, wpb_ref[...]) + bp_ref[...],
        0.0)
    a_ref[...] = _dot(xn, q_ref[...]) + qb_ref[...]
    pxn_ref[...] = jnp.maximum(_dot(xn, wu_ref[...]) + bu_ref[...], 0.0)


_mid = pl.pallas_call(
    _mid_body,
    grid=(NB,),
    in_specs=[
        pl.BlockSpec((BN, H), lambda i: (i, 0)),
        pl.BlockSpec((2, BN, H), lambda i: (0, i, 0)),
        pl.BlockSpec((H, H), lambda i: (0, 0)),
        pl.BlockSpec((H, H), lambda i: (0, 0)),
        pl.BlockSpec((1, H), lambda i: (0, 0)),
        pl.BlockSpec((H, H), lambda i: (0, 0)),
        pl.BlockSpec((1, H), lambda i: (0, 0)),
        pl.BlockSpec((H, H), lambda i: (0, 0)),
        pl.BlockSpec((1, H), lambda i: (0, 0)),
    ],
    out_specs=[pl.BlockSpec((BN, H), lambda i: (i, 0))] * 2,
    out_shape=[jax.ShapeDtypeStruct((N, H), jnp.float32)] * 2,
)


def _post_body(px_ref, agg_ref, wpt_ref, wpb_ref, bp_ref, batch_ref,
               w1_ref, b1_ref, w2_ref, b2_ref, y_ref, sums, cnt):
    i = pl.program_id(0)

    @pl.when(i == 0)
    def _():
        sums[...] = jnp.zeros_like(sums)
        cnt[...] = jnp.zeros_like(cnt)

    agg = agg_ref[0] + agg_ref[1]
    x3 = jnp.maximum(
        _dot(px_ref[...], wpt_ref[...]) + _dot(agg, wpb_ref[...]) + bp_ref[...],
        0.0)
    bvals = batch_ref[0]
    gids = lax.broadcasted_iota(jnp.int32, (G, BN), 0)
    oh_t = (gids == bvals).astype(jnp.float32)
    sums[...] += _dot(oh_t, x3)
    cnt[...] += jnp.broadcast_to(
        jnp.sum(oh_t, axis=1, keepdims=True), (G, H))

    @pl.when(i == NB - 1)
    def _():
        g = sums[...] / jnp.maximum(cnt[...], 1.0)
        h = jnp.maximum(_dot(g, w1_ref[...]) + b1_ref[...], 0.0)
        y_ref[...] = _dot(h, w2_ref[...]) + b2_ref[...]


_post = pl.pallas_call(
    _post_body,
    grid=(NB,),
    in_specs=[
        pl.BlockSpec((BN, H), lambda i: (i, 0)),
        pl.BlockSpec((2, BN, H), lambda i: (0, i, 0)),
        pl.BlockSpec((H, H), lambda i: (0, 0)),
        pl.BlockSpec((H, H), lambda i: (0, 0)),
        pl.BlockSpec((1, H), lambda i: (0, 0)),
        pl.BlockSpec((1, 1, BN), lambda i: (i, 0, 0)),
        pl.BlockSpec((H, H // 2), lambda i: (0, 0)),
        pl.BlockSpec((1, H // 2), lambda i: (0, 0)),
        pl.BlockSpec((H // 2, 1), lambda i: (0, 0)),
        pl.BlockSpec((1, 1), lambda i: (0, 0)),
    ],
    out_specs=pl.BlockSpec((G, 1), lambda i: (0, 0)),
    out_shape=jax.ShapeDtypeStruct((G, 1), jnp.float32),
    scratch_shapes=[
        pltpu.VMEM((G, H), jnp.float32),
        pltpu.VMEM((G, H), jnp.float32),
    ],
)


_sc_mesh = plsc.VectorSubcoreMesh(core_axis_name="c", subcore_axis_name="s")


@functools.partial(
    pl.kernel,
    mesh=_sc_mesh,
    out_type=jax.ShapeDtypeStruct((2, N, H), jnp.float32),
    scratch_types=[
        pltpu.VMEM((2, CH), jnp.int32),
        pltpu.VMEM((4, CH), jnp.int32),
        pltpu.VMEM((CH, H), jnp.float32),
        pltpu.VMEM((CH, H), jnp.float32),
        pltpu.VMEM((CH, H), jnp.float32),
        pltpu.VMEM((CH, H), jnp.float32),
        pltpu.VMEM((CH, H), jnp.float32),
        pltpu.VMEM((CH, H), jnp.float32),
        pltpu.VMEM_SHARED((N, H), jnp.float32),
        pltpu.SemaphoreType.DMA,
        pltpu.SemaphoreType.DMA,
        pltpu.SemaphoreType.DMA,
        pltpu.SemaphoreType.DMA,
        pltpu.SemaphoreType.DMA,
        pltpu.SemaphoreType.DMA,
        pltpu.SemaphoreType.DMA,
        pltpu.SemaphoreType.DMA,
        pltpu.SemaphoreType.DMA,
        pltpu.SemaphoreType.DMA,
        pltpu.SemaphoreType.DMA,
        pltpu.SemaphoreType.DMA,
    ],
)
def _sc_edge_pass(a_hbm, b_hbm, src_hbm, dst_hbm, out_hbm,
                  sidx, didx, rows0, rows1, bbuf0, bbuf1, mbuf0, mbuf1,
                  agg_sh,
                  gsem0, gsem1, bsem0, bsem1, ssem0, ssem1,
                  xsem0, xsem1, dsem0, dsem1, dsem2, dsem3):
    c = lax.axis_index("c")
    s = lax.axis_index("s")
    wid = c * 16 + s
    eb = wid * EPT

    rows = (rows0, rows1)
    bbuf = (bbuf0, bbuf1)
    mbuf = (mbuf0, mbuf1)
    gsem = (gsem0, gsem1)
    bsem = (bsem0, bsem1)
    ssem = (ssem0, ssem1)
    xsem = (xsem0, xsem1)
    dsem = (dsem0, dsem1, dsem2, dsem3)

    zero16 = jnp.zeros((16,), jnp.float32)

    @pl.loop(0, CH)
    def _(r):
        for j in range(H // 16):
            mbuf0[r, pl.ds(j * 16, 16)] = zero16

    for t in range(SB // CH):
        pltpu.sync_copy(mbuf0, agg_sh.at[pl.ds(s * SB + t * CH, CH)])
    pltpu.sync_copy(mbuf0.at[pl.ds(0, SB - (SB // CH) * CH)],
                    agg_sh.at[pl.ds(s * SB + (SB // CH) * CH,
                                    SB - (SB // CH) * CH)])

    @pl.when(s == 15)
    def _():
        pltpu.sync_copy(mbuf0.at[pl.ds(0, TAIL)],
                        agg_sh.at[pl.ds(16 * SB, TAIL)])

    plsc.subcore_barrier()

    def _load_sidx(b, k):
        pltpu.async_copy(src_hbm.at[wid, k], sidx.at[b], xsem[b])

    def _load_didx(d, k):
        pltpu.async_copy(dst_hbm.at[wid, k], didx.at[d], dsem[d])

    def _step(k, b, d):
        nb = 1 - b
        pltpu.make_async_copy(a_hbm.at[sidx.at[b]], rows[b], gsem[b]).wait()
        pltpu.make_async_copy(b_hbm.at[pl.ds(eb + k * CH, CH)],
                              bbuf[b], bsem[b]).wait()

        @pl.when(k + 2 < NCH)
        def _():
            _load_sidx(b, k + 2)

        @pl.when(k + 1 < NCH)
        def _():
            pltpu.make_async_copy(src_hbm.at[wid, k + 1], sidx.at[nb],
                                  xsem[nb]).wait()
            pltpu.async_copy(a_hbm.at[sidx.at[nb]], rows[nb], gsem[nb])
            pltpu.async_copy(b_hbm.at[pl.ds(eb + (k + 1) * CH, CH)],
                             bbuf[nb], bsem[nb])

        @pl.when(k >= 2)
        def _():
            pltpu.make_async_copy(mbuf[b], agg_sh.at[didx.at[d]],
                                  ssem[b]).wait()

        @pl.when(k + 2 < NCH)
        def _():
            _load_didx((d + 2) % 4, k + 2)

        pltpu.make_async_copy(dst_hbm.at[wid, k], didx.at[d],
                              dsem[d]).wait()

        @pl.loop(0, CH // 2)
        def _(e2):
            e = e2 * 2
            for j in range(H // 16):
                sl = pl.ds(j * 16, 16)
                mbuf[b][e, sl] = jnp.maximum(
                    rows[b][e, sl] + bbuf[b][e, sl], 0.0)
            for j in range(H // 16):
                sl = pl.ds(j * 16, 16)
                mbuf[b][e + 1, sl] = jnp.maximum(
                    rows[b][e + 1, sl] + bbuf[b][e + 1, sl], 0.0)

        pltpu.async_copy(mbuf[b], agg_sh.at[didx.at[d]], ssem[b], add=True)

    _load_sidx(0, 0)
    _load_didx(0, 0)
    _load_didx(1, 1)
    pltpu.make_async_copy(src_hbm.at[wid, 0], sidx.at[0], xsem[0]).wait()
    pltpu.async_copy(a_hbm.at[sidx.at[0]], rows[0], gsem[0])
    pltpu.async_copy(b_hbm.at[pl.ds(eb, CH)], bbuf[0], bsem[0])
    _load_sidx(1, 1)

    _step(0, 0, 0)
    _step(1, 1, 1)

    @pl.loop(2, NCH, step=4)
    def _(k):
        _step(k, 0, 2)
        _step(k + 1, 1, 3)
        _step(k + 2, 0, 0)
        _step(k + 3, 1, 1)

    pltpu.make_async_copy(mbuf[0], agg_sh.at[didx.at[0]], ssem[0]).wait()
    pltpu.make_async_copy(mbuf[1], agg_sh.at[didx.at[1]], ssem[1]).wait()

    plsc.subcore_barrier()
    pltpu.sync_copy(agg_sh.at[pl.ds(s * SB, SB)],
                    out_hbm.at[c, pl.ds(s * SB, SB)])

    @pl.when(s == 15)
    def _():
        pltpu.sync_copy(agg_sh.at[pl.ds(16 * SB, TAIL)],
                        out_hbm.at[c, pl.ds(16 * SB, TAIL)])


def kernel(x, edge_index, edge_attr, batch,
           Ws1, bs1, We1, be1, Wm1, bm1, Wu1, bu1, Wp1, bp1,
           Ws2, bs2, We2, be2, Wm2, bm2, Wu2, bu2, Wp2, bp2,
           Ws3, bs3, We3, be3, Wm3, bm3, Wu3, bu3, Wp3, bp3,
           W1, b1, W2, b2):
    src = edge_index[0].reshape(NTILES, NCH, CH)
    dst = edge_index[1].reshape(NTILES, NCH, CH)

    layers = [
        (Ws1, bs1, We1, be1, Wm1, bm1, Wu1, bu1, Wp1, bp1),
        (Ws2, bs2, We2, be2, Wm2, bm2, Wu2, bu2, Wp2, bp2),
        (Ws3, bs3, We3, be3, Wm3, bm3, Wu3, bu3, Wp3, bp3),
    ]

    Q, qb, C, dv, Wpt, Wpb, bpv, Wu_, bu_ = [], [], [], [], [], [], [], [], []
    for (Ws, bs, We, be, Wm, bm, Wu, bu, Wp, bp) in layers:
        Wmt, Wmb = Wm[:H], Wm[H:]
        Q.append(Ws @ Wmt)
        qb.append((bs @ Wmt).reshape(1, H))
        C.append(We @ Wmb)
        dv.append((be @ Wmb + bm).reshape(1, H))
        Wpt.append(Wp[:H])
        Wpb.append(Wp[H:])
        bpv.append(bp.reshape(1, H))
        Wu_.append(Wu)
        bu_.append(bu.reshape(1, H))

    B1 = _edge_b1(edge_attr, C[0], dv[0])
    A, px = _pre(x, Q[0], qb[0], Wu_[0], bu_[0])
    Bs = [B1, None, None]
    for l in range(3):
        agg2 = _sc_edge_pass(A, Bs[l], src, dst)
        if l == 0:
            B2, B3 = _edge_b23(edge_attr, jnp.stack(C[1:]), jnp.stack(dv[1:]))
            Bs[1], Bs[2] = B2, B3
        if l < 2:
            A, px = _mid(px, agg2, Wpt[l], Wpb[l], bpv[l],
                         Q[l + 1], qb[l + 1], Wu_[l + 1], bu_[l + 1])
        else:
            y = _post(px, agg2, Wpt[l], Wpb[l], bpv[l],
                      batch.reshape(NB, 1, BN),
                      W1, b1.reshape(1, H // 2), W2, b2.reshape(1, 1))
    return y


# speedup vs baseline: 1.0276x; 1.0276x over previous
import functools

import jax
import jax.numpy as jnp
import numpy as np
from jax import lax
from jax.experimental import pallas as pl
from jax.experimental.pallas import tpu as pltpu
from jax.experimental.pallas import tpu_sc as plsc

N = 10000
E = 320000
D = 128
DE = 16
H = 128
G = 64

NB = 10
BN = N // NB
EB = 40
BE = E // EB

NTILES = 32
EPT = E // NTILES
CH = 40
NCH = EPT // CH
SB = 624
TAIL = N - 16 * SB
ZR = 104


def _dot(a, b):
    return jnp.dot(a, b, preferred_element_type=jnp.float32)


def _edge_b1_body(ea_ref, c_ref, d_ref, b1_ref):
    b1_ref[...] = _dot(ea_ref[...], c_ref[...]) + d_ref[...]


_edge_b1 = pl.pallas_call(
    _edge_b1_body,
    grid=(EB,),
    in_specs=[
        pl.BlockSpec((BE, DE), lambda i: (i, 0)),
        pl.BlockSpec((DE, H), lambda i: (0, 0)),
        pl.BlockSpec((1, H), lambda i: (0, 0)),
    ],
    out_specs=pl.BlockSpec((BE, H), lambda i: (i, 0)),
    out_shape=jax.ShapeDtypeStruct((E, H), jnp.float32),
)


def _edge_b23_body(ea_ref, c_ref, d_ref, b2_ref, b3_ref):
    ea = ea_ref[...]
    b2_ref[...] = _dot(ea, c_ref[0]) + d_ref[0]
    b3_ref[...] = _dot(ea, c_ref[1]) + d_ref[1]


_edge_b23 = pl.pallas_call(
    _edge_b23_body,
    grid=(EB,),
    in_specs=[
        pl.BlockSpec((BE, DE), lambda i: (i, 0)),
        pl.BlockSpec((2, DE, H), lambda i: (0, 0, 0)),
        pl.BlockSpec((2, 1, H), lambda i: (0, 0, 0)),
    ],
    out_specs=[pl.BlockSpec((BE, H), lambda i: (i, 0))] * 2,
    out_shape=[jax.ShapeDtypeStruct((E, H), jnp.float32)] * 2,
)


def _pre_body(x_ref, q_ref, qb_ref, wu_ref, bu_ref, a_ref, px_ref):
    x = x_ref[...]
    a_ref[...] = _dot(x, q_ref[...]) + qb_ref[...]
    px_ref[...] = jnp.maximum(_dot(x, wu_ref[...]) + bu_ref[...], 0.0)


_pre = pl.pallas_call(
    _pre_body,
    grid=(NB,),
    in_specs=[
        pl.BlockSpec((BN, D), lambda i: (i, 0)),
        pl.BlockSpec((D, H), lambda i: (0, 0)),
        pl.BlockSpec((1, H), lambda i: (0, 0)),
        pl.BlockSpec((D, H), lambda i: (0, 0)),
        pl.BlockSpec((1, H), lambda i: (0, 0)),
    ],
    out_specs=[pl.BlockSpec((BN, H), lambda i: (i, 0))] * 2,
    out_shape=[jax.ShapeDtypeStruct((N, H), jnp.float32)] * 2,
)


def _mid_body(px_ref, agg_ref, wpt_ref, wpb_ref, bp_ref,
              q_ref, qb_ref, wu_ref, bu_ref, a_ref, pxn_ref):
    agg = agg_ref[0] + agg_ref[1]
    xn = jnp.maximum(
        _dot(px_ref[...], wpt_ref[...]) + _dot(agg, wpb_ref[...]) + bp_ref[...],
        0.0)
    a_ref[...] = _dot(xn, q_ref[...]) + qb_ref[...]
    pxn_ref[...] = jnp.maximum(_dot(xn, wu_ref[...]) + bu_ref[...], 0.0)


_mid = pl.pallas_call(
    _mid_body,
    grid=(NB,),
    in_specs=[
        pl.BlockSpec((BN, H), lambda i: (i, 0)),
        pl.BlockSpec((2, BN, H), lambda i: (0, i, 0)),
        pl.BlockSpec((H, H), lambda i: (0, 0)),
        pl.BlockSpec((H, H), lambda i: (0, 0)),
        pl.BlockSpec((1, H), lambda i: (0, 0)),
        pl.BlockSpec((H, H), lambda i: (0, 0)),
        pl.BlockSpec((1, H), lambda i: (0, 0)),
        pl.BlockSpec((H, H), lambda i: (0, 0)),
        pl.BlockSpec((1, H), lambda i: (0, 0)),
    ],
    out_specs=[pl.BlockSpec((BN, H), lambda i: (i, 0))] * 2,
    out_shape=[jax.ShapeDtypeStruct((N, H), jnp.float32)] * 2,
)


def _post_body(px_ref, agg_ref, wpt_ref, wpb_ref, bp_ref, batch_ref,
               w1_ref, b1_ref, w2_ref, b2_ref, y_ref, sums, cnt):
    i = pl.program_id(0)

    @pl.when(i == 0)
    def _():
        sums[...] = jnp.zeros_like(sums)
        cnt[...] = jnp.zeros_like(cnt)

    agg = agg_ref[0] + agg_ref[1]
    x3 = jnp.maximum(
        _dot(px_ref[...], wpt_ref[...]) + _dot(agg, wpb_ref[...]) + bp_ref[...],
        0.0)
    bvals = batch_ref[0]
    gids = lax.broadcasted_iota(jnp.int32, (G, BN), 0)
    oh_t = (gids == bvals).astype(jnp.float32)
    sums[...] += _dot(oh_t, x3)
    cnt[...] += jnp.broadcast_to(
        jnp.sum(oh_t, axis=1, keepdims=True), (G, H))

    @pl.when(i == NB - 1)
    def _():
        g = sums[...] / jnp.maximum(cnt[...], 1.0)
        h = jnp.maximum(_dot(g, w1_ref[...]) + b1_ref[...], 0.0)
        y_ref[...] = _dot(h, w2_ref[...]) + b2_ref[...]


_post = pl.pallas_call(
    _post_body,
    grid=(NB,),
    in_specs=[
        pl.BlockSpec((BN, H), lambda i: (i, 0)),
        pl.BlockSpec((2, BN, H), lambda i: (0, i, 0)),
        pl.BlockSpec((H, H), lambda i: (0, 0)),
        pl.BlockSpec((H, H), lambda i: (0, 0)),
        pl.BlockSpec((1, H), lambda i: (0, 0)),
        pl.BlockSpec((1, 1, BN), lambda i: (i, 0, 0)),
        pl.BlockSpec((H, H // 2), lambda i: (0, 0)),
        pl.BlockSpec((1, H // 2), lambda i: (0, 0)),
        pl.BlockSpec((H // 2, 1), lambda i: (0, 0)),
        pl.BlockSpec((1, 1), lambda i: (0, 0)),
    ],
    out_specs=pl.BlockSpec((G, 1), lambda i: (0, 0)),
    out_shape=jax.ShapeDtypeStruct((G, 1), jnp.float32),
    scratch_shapes=[
        pltpu.VMEM((G, H), jnp.float32),
        pltpu.VMEM((G, H), jnp.float32),
    ],
)


_sc_mesh = plsc.VectorSubcoreMesh(core_axis_name="c", subcore_axis_name="s")


@functools.partial(
    pl.kernel,
    mesh=_sc_mesh,
    out_type=jax.ShapeDtypeStruct((2, N, H), jnp.float32),
    scratch_types=[
        pltpu.VMEM((4, CH), jnp.int32),
        pltpu.VMEM((4, CH), jnp.int32),
        pltpu.VMEM((CH, H), jnp.float32),
        pltpu.VMEM((CH, H), jnp.float32),
        pltpu.VMEM((CH, H), jnp.float32),
        pltpu.VMEM((CH, H), jnp.float32),
        pltpu.VMEM((CH, H), jnp.float32),
        pltpu.VMEM((CH, H), jnp.float32),
        pltpu.VMEM((CH, H), jnp.float32),
        pltpu.VMEM((CH, H), jnp.float32),
        pltpu.VMEM_SHARED((N, H), jnp.float32),
        pltpu.SemaphoreType.DMA,
        pltpu.SemaphoreType.DMA,
        pltpu.SemaphoreType.DMA,
        pltpu.SemaphoreType.DMA,
        pltpu.SemaphoreType.DMA,
        pltpu.SemaphoreType.DMA,
        pltpu.SemaphoreType.DMA,
        pltpu.SemaphoreType.DMA,
        pltpu.SemaphoreType.DMA,
        pltpu.SemaphoreType.DMA,
        pltpu.SemaphoreType.DMA,
        pltpu.SemaphoreType.DMA,
        pltpu.SemaphoreType.DMA,
        pltpu.SemaphoreType.DMA,
        pltpu.SemaphoreType.DMA,
        pltpu.SemaphoreType.DMA,
    ],
)
def _sc_edge_pass(a_hbm, b_hbm, src_hbm, dst_hbm, out_hbm,
                  sidx, didx, rows0, rows1, rows2, rows3,
                  bbuf0, bbuf1, mbuf0, mbuf1, agg_sh,
                  gsem0, gsem1, gsem2, gsem3, bsem0, bsem1, ssem0, ssem1,
                  xsem0, xsem1, xsem2, xsem3, dsem0, dsem1, dsem2, dsem3):
    c = lax.axis_index("c")
    s = lax.axis_index("s")
    wid = c * 16 + s
    eb = wid * EPT

    rows = (rows0, rows1, rows2, rows3)
    bbuf = (bbuf0, bbuf1)
    mbuf = (mbuf0, mbuf1)
    gsem = (gsem0, gsem1, gsem2, gsem3)
    bsem = (bsem0, bsem1)
    ssem = (ssem0, ssem1)
    xsem = (xsem0, xsem1, xsem2, xsem3)
    dsem = (dsem0, dsem1, dsem2, dsem3)

    zero16 = jnp.zeros((16,), jnp.float32)

    @pl.loop(0, CH)
    def _(r):
        for j in range(H // 16):
            mbuf0[r, pl.ds(j * 16, 16)] = zero16

    for t in range(SB // CH):
        pltpu.sync_copy(mbuf0, agg_sh.at[pl.ds(s * SB + t * CH, CH)])
    pltpu.sync_copy(mbuf0.at[pl.ds(0, SB - (SB // CH) * CH)],
                    agg_sh.at[pl.ds(s * SB + (SB // CH) * CH,
                                    SB - (SB // CH) * CH)])

    @pl.when(s == 15)
    def _():
        pltpu.sync_copy(mbuf0.at[pl.ds(0, TAIL)],
                        agg_sh.at[pl.ds(16 * SB, TAIL)])

    plsc.subcore_barrier()

    def _load_sidx(r, k):
        pltpu.async_copy(src_hbm.at[wid, k], sidx.at[r], xsem[r])

    def _load_didx(d, k):
        pltpu.async_copy(dst_hbm.at[wid, k], didx.at[d], dsem[d])

    def _gather(r, k):
        pltpu.async_copy(a_hbm.at[sidx.at[r]], rows[r], gsem[r])

    def _step(k, b, r):
        nb = 1 - b
        pltpu.make_async_copy(a_hbm.at[sidx.at[r]], rows[r], gsem[r]).wait()
        pltpu.make_async_copy(b_hbm.at[pl.ds(eb + k * CH, CH)],
                              bbuf[b], bsem[b]).wait()

        @pl.when(k + 3 < NCH)
        def _():
            _load_sidx((r + 3) % 4, k + 3)

        @pl.when(k + 2 < NCH)
        def _():
            pltpu.make_async_copy(src_hbm.at[wid, k + 2],
                                  sidx.at[(r + 2) % 4],
                                  xsem[(r + 2) % 4]).wait()
            _gather((r + 2) % 4, k + 2)

        @pl.when(k + 1 < NCH)
        def _():
            pltpu.async_copy(b_hbm.at[pl.ds(eb + (k + 1) * CH, CH)],
                             bbuf[nb], bsem[nb])

        @pl.when(k >= 2)
        def _():
            pltpu.make_async_copy(mbuf[b], agg_sh.at[didx.at[r]],
                                  ssem[b]).wait()

        @pl.when(k + 2 < NCH)
        def _():
            _load_didx((r + 2) % 4, k + 2)

        pltpu.make_async_copy(dst_hbm.at[wid, k], didx.at[r],
                              dsem[r]).wait()

        @pl.loop(0, CH)
        def _(e):
            for j in range(H // 16):
                sl = pl.ds(j * 16, 16)
                mbuf[b][e, sl] = jnp.maximum(
                    rows[r][e, sl] + bbuf[b][e, sl], 0.0)

        pltpu.async_copy(mbuf[b], agg_sh.at[didx.at[r]], ssem[b], add=True)

    _load_sidx(0, 0)
    _load_sidx(1, 1)
    _load_sidx(2, 2)
    _load_didx(0, 0)
    _load_didx(1, 1)
    pltpu.make_async_copy(src_hbm.at[wid, 0], sidx.at[0], xsem[0]).wait()
    _gather(0, 0)
    pltpu.make_async_copy(src_hbm.at[wid, 1], sidx.at[1], xsem[1]).wait()
    _gather(1, 1)
    pltpu.async_copy(b_hbm.at[pl.ds(eb, CH)], bbuf[0], bsem[0])

    _step(0, 0, 0)
    _step(1, 1, 1)

    @pl.loop(2, NCH, step=4)
    def _(k):
        _step(k, 0, 2)
        _step(k + 1, 1, 3)
        _step(k + 2, 0, 0)
        _step(k + 3, 1, 1)

    pltpu.make_async_copy(mbuf[0], agg_sh.at[didx.at[0]], ssem[0]).wait()
    pltpu.make_async_copy(mbuf[1], agg_sh.at[didx.at[1]], ssem[1]).wait()

    plsc.subcore_barrier()
    pltpu.sync_copy(agg_sh.at[pl.ds(s * SB, SB)],
                    out_hbm.at[c, pl.ds(s * SB, SB)])

    @pl.when(s == 15)
    def _():
        pltpu.sync_copy(agg_sh.at[pl.ds(16 * SB, TAIL)],
                        out_hbm.at[c, pl.ds(16 * SB, TAIL)])


def kernel(x, edge_index, edge_attr, batch,
           Ws1, bs1, We1, be1, Wm1, bm1, Wu1, bu1, Wp1, bp1,
           Ws2, bs2, We2, be2, Wm2, bm2, Wu2, bu2, Wp2, bp2,
           Ws3, bs3, We3, be3, Wm3, bm3, Wu3, bu3, Wp3, bp3,
           W1, b1, W2, b2):
    src = edge_index[0].reshape(NTILES, NCH, CH)
    dst = edge_index[1].reshape(NTILES, NCH, CH)

    layers = [
        (Ws1, bs1, We1, be1, Wm1, bm1, Wu1, bu1, Wp1, bp1),
        (Ws2, bs2, We2, be2, Wm2, bm2, Wu2, bu2, Wp2, bp2),
        (Ws3, bs3, We3, be3, Wm3, bm3, Wu3, bu3, Wp3, bp3),
    ]

    Q, qb, C, dv, Wpt, Wpb, bpv, Wu_, bu_ = [], [], [], [], [], [], [], [], []
    for (Ws, bs, We, be, Wm, bm, Wu, bu, Wp, bp) in layers:
        Wmt, Wmb = Wm[:H], Wm[H:]
        Q.append(Ws @ Wmt)
        qb.append((bs @ Wmt).reshape(1, H))
        C.append(We @ Wmb)
        dv.append((be @ Wmb + bm).reshape(1, H))
        Wpt.append(Wp[:H])
        Wpb.append(Wp[H:])
        bpv.append(bp.reshape(1, H))
        Wu_.append(Wu)
        bu_.append(bu.reshape(1, H))

    B1 = _edge_b1(edge_attr, C[0], dv[0])
    A, px = _pre(x, Q[0], qb[0], Wu_[0], bu_[0])
    Bs = [B1, None, None]
    for l in range(3):
        agg2 = _sc_edge_pass(A, Bs[l], src, dst)
        if l == 0:
            B2, B3 = _edge_b23(edge_attr, jnp.stack(C[1:]), jnp.stack(dv[1:]))
            Bs[1], Bs[2] = B2, B3
        if l < 2:
            A, px = _mid(px, agg2, Wpt[l], Wpb[l], bpv[l],
                         Q[l + 1], qb[l + 1], Wu_[l + 1], bu_[l + 1])
        else:
            y = _post(px, agg2, Wpt[l], Wpb[l], bpv[l],
                      batch.reshape(NB, 1, BN),
                      W1, b1.reshape(1, H // 2), W2, b2.reshape(1, 1))
    return y
